# baseline (speedup 1.0000x reference)
import functools

import jax
import jax.numpy as jnp
from jax import lax
from jax.experimental import pallas as pl
from jax.experimental.pallas import tpu as pltpu
from jax.experimental.pallas import tpu_sc as plsc

B = 1024
HIST = 50
NEG = 10
D = 64

NC = 2
NS = 16
NW = NC * NS

H_TOT = B * HIST
N_TOT = B * NEG
H_PW = H_TOT // NW
N_PW = N_TOT // NW
B_PW = B // NW
H_CHUNK = 400
H_NCH = H_PW // H_CHUNK


def _gather_body(emb_hbm, dtab_hbm, sidx_hbm, tidx_hbm, hidx_hbm, nidx_hbm,
                 s_out, t_out, h_out, n_out, d_out,
                 idx_h, rows_h, idx_n, rows_n, idx_b, rows_b, rows_d, sem):
    wid = lax.axis_index("s") * NC + lax.axis_index("c")

    for ci in range(H_NCH):
        off = wid * H_PW + ci * H_CHUNK
        pltpu.sync_copy(hidx_hbm.at[pl.ds(off, H_CHUNK)], idx_h)
        pltpu.async_copy(emb_hbm.at[idx_h], rows_h, sem).wait()
        pltpu.sync_copy(rows_h, h_out.at[pl.ds(off, H_CHUNK)])

    offn = wid * N_PW
    pltpu.sync_copy(nidx_hbm.at[pl.ds(offn, N_PW)], idx_n)
    pltpu.async_copy(emb_hbm.at[idx_n], rows_n, sem).wait()
    pltpu.sync_copy(rows_n, n_out.at[pl.ds(offn, N_PW)])

    offb = wid * B_PW
    pltpu.sync_copy(sidx_hbm.at[pl.ds(offb, B_PW)], idx_b)
    pltpu.async_copy(emb_hbm.at[idx_b], rows_b, sem).wait()
    pltpu.sync_copy(rows_b, s_out.at[pl.ds(offb, B_PW)])
    pltpu.async_copy(dtab_hbm.at[idx_b], rows_d, sem).wait()
    pltpu.sync_copy(rows_d, d_out.at[pl.ds(offb, B_PW)])

    pltpu.sync_copy(tidx_hbm.at[pl.ds(offb, B_PW)], idx_b)
    pltpu.async_copy(emb_hbm.at[idx_b], rows_b, sem).wait()
    pltpu.sync_copy(rows_b, t_out.at[pl.ds(offb, B_PW)])


def _sc_gather(node_emb, delta_tab, s_idx, t_idx, h_idx, n_idx):
    mesh = plsc.VectorSubcoreMesh(core_axis_name="c", subcore_axis_name="s")
    f = functools.partial(
        pl.kernel,
        mesh=mesh,
        out_type=[
            jax.ShapeDtypeStruct((B, D), jnp.float32),
            jax.ShapeDtypeStruct((B, D), jnp.float32),
            jax.ShapeDtypeStruct((H_TOT, D), jnp.float32),
            jax.ShapeDtypeStruct((N_TOT, D), jnp.float32),
            jax.ShapeDtypeStruct((B, 1), jnp.float32),
        ],
        scratch_types=[
            pltpu.VMEM((H_CHUNK,), jnp.int32),
            pltpu.VMEM((H_CHUNK, D), jnp.float32),
            pltpu.VMEM((N_PW,), jnp.int32),
            pltpu.VMEM((N_PW, D), jnp.float32),
            pltpu.VMEM((B_PW,), jnp.int32),
            pltpu.VMEM((B_PW, D), jnp.float32),
            pltpu.VMEM((B_PW, 1), jnp.float32),
            pltpu.SemaphoreType.DMA,
        ],
        compiler_params=pltpu.CompilerParams(use_tc_tiling_on_sc=False),
    )(_gather_body)
    return f(node_emb, delta_tab, s_idx, t_idx, h_idx, n_idx)


BB = 128
GRID = B // BB


def _tc_body(s_ref, t_ref, h_ref, n_ref, de_ref, tt_ref, ht_ref, hm_ref,
             out_ref):
    s = s_ref[...]
    t = t_ref[...]
    h = h_ref[...]
    nn = n_ref[...]
    delta = de_ref[...]
    tt = tt_ref[...]
    ht = ht_ref[...]
    hm = hm_ref[...]

    d2_sh = jnp.sum((s[:, None, :] - h) ** 2, axis=2)
    att = jax.nn.softmax(-d2_sh, axis=1)
    c = att * jnp.exp(delta * jnp.abs(tt - ht)) * hm
    c0 = jnp.sum(c, axis=1, keepdims=True)
    h2 = jnp.sum(h * h, axis=2)
    c1 = jnp.sum(c * h2, axis=1, keepdims=True)
    hbar = jnp.sum(c[:, :, None] * h, axis=1)

    p_mu = -jnp.sum((s - t) ** 2, axis=1, keepdims=True)
    t2 = jnp.sum(t * t, axis=1, keepdims=True)
    ht_dot = jnp.sum(hbar * t, axis=1, keepdims=True)
    p_lam = p_mu - c1 + 2.0 * ht_dot - c0 * t2

    n_mu = -jnp.sum((s[:, None, :] - nn) ** 2, axis=2)
    n2 = jnp.sum(nn * nn, axis=2)
    hn_dot = jnp.sum(hbar[:, None, :] * nn, axis=2)
    n_lam = n_mu - c1 + 2.0 * hn_dot - c0 * n2

    pos = -jnp.log(jax.nn.sigmoid(p_lam) + 1e-6)
    neg = jnp.sum(jnp.log(jax.nn.sigmoid(-n_lam) + 1e-6),
                  axis=1, keepdims=True)
    out_ref[...] = pos - neg


def _tc_compute(s_emb, t_emb, h3, n3, delta, t_times, h_times, h_mask):
    return pl.pallas_call(
        _tc_body,
        grid=(GRID,),
        in_specs=[
            pl.BlockSpec((BB, D), lambda i: (i, 0)),
            pl.BlockSpec((BB, D), lambda i: (i, 0)),
            pl.BlockSpec((BB, HIST, D), lambda i: (i, 0, 0)),
            pl.BlockSpec((BB, NEG, D), lambda i: (i, 0, 0)),
            pl.BlockSpec((BB, 1), lambda i: (i, 0)),
            pl.BlockSpec((BB, 1), lambda i: (i, 0)),
            pl.BlockSpec((BB, HIST), lambda i: (i, 0)),
            pl.BlockSpec((BB, HIST), lambda i: (i, 0)),
        ],
        out_specs=pl.BlockSpec((BB, 1), lambda i: (i, 0)),
        out_shape=jax.ShapeDtypeStruct((B, 1), jnp.float32),
    )(s_emb, t_emb, h3, n3, delta, t_times, h_times, h_mask)


def kernel(s_nodes, t_nodes, t_times, h_nodes, h_times, h_time_mask,
           n_nodes, node_emb, delta_tab):
    s_idx = s_nodes.reshape(B)
    t_idx = t_nodes.reshape(B)
    h_idx = h_nodes.reshape(H_TOT)
    n_idx = n_nodes.reshape(N_TOT)
    s_emb, t_emb, h_emb, n_emb, delta = _sc_gather(
        node_emb, delta_tab, s_idx, t_idx, h_idx, n_idx)
    out = _tc_compute(s_emb, t_emb,
                      h_emb.reshape(B, HIST, D), n_emb.reshape(B, NEG, D),
                      delta, t_times, h_times, h_time_mask)
    return out.reshape(B)


# speedup vs baseline: 1.9751x; 1.9751x over previous
import functools

import jax
import jax.numpy as jnp
from jax import lax
from jax.experimental import pallas as pl
from jax.experimental.pallas import tpu as pltpu
from jax.experimental.pallas import tpu_sc as plsc

B = 1024
HIST = 50
NEG = 10
D = 64

NC = 2
NS = 16
NW = NC * NS

H_TOT = B * HIST
N_TOT = B * NEG
IDX_TOT = H_TOT + N_TOT + B + B
S_OFF = H_TOT + N_TOT
IDX_PAD = 65536

PW = IDX_PAD // NW
CH = 256
NCH = PW // CH
GL = 16
DCH = B // NW


def _gather_body(emb_hbm, dtab_hbm, idx_hbm, rows_out, delta_out,
                 idx_v, rows0, rows1, drows,
                 sem_g0, sem_g1, sem_o):
    wid = lax.axis_index("s") * NC + lax.axis_index("c")
    base = wid * PW
    bufs = (rows0, rows1)
    sems = (sem_g0, sem_g1)

    pltpu.sync_copy(idx_hbm.at[pl.ds(base, PW)], idx_v)

    for c in range(NCH + 1):
        if c < NCH:
            buf = bufs[c % 2]
            if c >= 2:
                pltpu.make_async_copy(
                    rows_out.at[pl.ds(0, CH)], buf, sem_o).wait()

            def issue(g, _, c=c, buf=buf, sem=sems[c % 2]):
                vec = idx_v[pl.ds(c * CH + g * GL, GL)]
                for k in range(GL):
                    i = vec[k]
                    pltpu.make_async_copy(
                        emb_hbm.at[pl.ds(i, 1), :],
                        buf.at[pl.ds(g * GL + k, 1), :], sem).start()
                return 0

            lax.fori_loop(0, CH // GL, issue, 0)
        if c >= 1:
            pbuf = bufs[(c - 1) % 2]
            pltpu.make_async_copy(
                rows_out.at[pl.ds(0, CH)], pbuf, sems[(c - 1) % 2]).wait()
            pltpu.make_async_copy(
                pbuf, rows_out.at[pl.ds(base + (c - 1) * CH, CH)],
                sem_o).start()

    pltpu.make_async_copy(rows_out.at[pl.ds(0, CH)], rows0, sem_o).wait()
    pltpu.make_async_copy(rows_out.at[pl.ds(0, CH)], rows1, sem_o).wait()

    doff = wid * DCH
    pltpu.sync_copy(idx_hbm.at[pl.ds(S_OFF + doff, DCH)], idx_v.at[pl.ds(0, DCH)])

    def issue_d(g, _):
        vec = idx_v[pl.ds(g * GL, GL)]
        for k in range(GL):
            i = vec[k]
            pltpu.make_async_copy(
                dtab_hbm.at[pl.ds(i, 1), :],
                drows.at[pl.ds(g * GL + k, 1), :], sem_g0).start()
        return 0

    lax.fori_loop(0, DCH // GL, issue_d, 0)
    pltpu.make_async_copy(delta_out.at[pl.ds(0, DCH)], drows, sem_g0).wait()
    pltpu.sync_copy(drows, delta_out.at[pl.ds(doff, DCH)])


def _sc_gather(node_emb, delta_tab, idx_all):
    mesh = plsc.VectorSubcoreMesh(core_axis_name="c", subcore_axis_name="s")
    f = functools.partial(
        pl.kernel,
        mesh=mesh,
        out_type=[
            jax.ShapeDtypeStruct((IDX_PAD, D), jnp.float32),
            jax.ShapeDtypeStruct((B, 1), jnp.float32),
        ],
        scratch_types=[
            pltpu.VMEM((PW,), jnp.int32),
            pltpu.VMEM((CH, D), jnp.float32),
            pltpu.VMEM((CH, D), jnp.float32),
            pltpu.VMEM((DCH, 1), jnp.float32),
            pltpu.SemaphoreType.DMA,
            pltpu.SemaphoreType.DMA,
            pltpu.SemaphoreType.DMA,
        ],
    )(_gather_body)
    return f(node_emb, delta_tab, idx_all)


BB = 128
GRID = B // BB


def _tc_body(h_ref, n_ref, s_ref, t_ref, de_ref, tt_ref, ht_ref, hm_ref,
             out_ref):
    s = s_ref[...]
    t = t_ref[...]
    h = h_ref[...].reshape(BB, HIST, D)
    nn = n_ref[...].reshape(BB, NEG, D)
    delta = de_ref[...]
    tt = tt_ref[...]
    ht = ht_ref[...]
    hm = hm_ref[...]

    d2_sh = jnp.sum((s[:, None, :] - h) ** 2, axis=2)
    att = jax.nn.softmax(-d2_sh, axis=1)
    c = att * jnp.exp(delta * jnp.abs(tt - ht)) * hm
    c0 = jnp.sum(c, axis=1, keepdims=True)
    h2 = jnp.sum(h * h, axis=2)
    c1 = jnp.sum(c * h2, axis=1, keepdims=True)
    hbar = jnp.sum(c[:, :, None] * h, axis=1)

    p_mu = -jnp.sum((s - t) ** 2, axis=1, keepdims=True)
    t2 = jnp.sum(t * t, axis=1, keepdims=True)
    ht_dot = jnp.sum(hbar * t, axis=1, keepdims=True)
    p_lam = p_mu - c1 + 2.0 * ht_dot - c0 * t2

    n_mu = -jnp.sum((s[:, None, :] - nn) ** 2, axis=2)
    n2 = jnp.sum(nn * nn, axis=2)
    hn_dot = jnp.sum(hbar[:, None, :] * nn, axis=2)
    n_lam = n_mu - c1 + 2.0 * hn_dot - c0 * n2

    pos = -jnp.log(jax.nn.sigmoid(p_lam) + 1e-6)
    neg = jnp.sum(jnp.log(jax.nn.sigmoid(-n_lam) + 1e-6),
                  axis=1, keepdims=True)
    out_ref[...] = pos - neg


def _tc_compute(rows, delta, t_times, h_times, h_mask):
    return pl.pallas_call(
        _tc_body,
        grid=(GRID,),
        in_specs=[
            pl.BlockSpec((BB * HIST, D), lambda i: (i, 0)),
            pl.BlockSpec((BB * NEG, D), lambda i: (H_TOT // (BB * NEG) + i, 0)),
            pl.BlockSpec((BB, D), lambda i: (S_OFF // BB + i, 0)),
            pl.BlockSpec((BB, D), lambda i: ((S_OFF + B) // BB + i, 0)),
            pl.BlockSpec((BB, 1), lambda i: (i, 0)),
            pl.BlockSpec((BB, 1), lambda i: (i, 0)),
            pl.BlockSpec((BB, HIST), lambda i: (i, 0)),
            pl.BlockSpec((BB, HIST), lambda i: (i, 0)),
        ],
        out_specs=pl.BlockSpec((BB, 1), lambda i: (i, 0)),
        out_shape=jax.ShapeDtypeStruct((B, 1), jnp.float32),
    )(rows, rows, rows, rows, delta, t_times, h_times, h_mask)


def kernel(s_nodes, t_nodes, t_times, h_nodes, h_times, h_time_mask,
           n_nodes, node_emb, delta_tab):
    idx_all = jnp.concatenate([
        h_nodes.reshape(H_TOT), n_nodes.reshape(N_TOT),
        s_nodes.reshape(B), t_nodes.reshape(B),
        jnp.zeros((IDX_PAD - IDX_TOT,), jnp.int32)])
    rows, delta = _sc_gather(node_emb, delta_tab, idx_all)
    out = _tc_compute(rows, delta, t_times, h_times, h_time_mask)
    return out.reshape(B)
